# Initial kernel scaffold; baseline (speedup 1.0000x reference)
import jax
import jax.numpy as jnp
from jax.experimental import pallas as pl


def kernel(x, edge_index, W0, b0, Wh, bh, Wf, bf):
    raise NotImplementedError("write your pallas kernel here")


# speedup vs baseline: 10.6702x; 10.6702x over previous
import functools

import jax
import jax.numpy as jnp
from jax import lax
from jax.experimental import pallas as pl
from jax.experimental.pallas import tpu as pltpu
from jax.experimental.pallas import tpu_sc as plsc

N = 100000
E = 3200000
NC = 2
NS = 16
CW = 80
CH = 10
EROWS = E // CW
ROWS_PER_SUB = EROWS // NS
WINS = ROWS_PER_SUB // CH
NZ = 125
NODE_ROWS_PER_SUB = N // NS


def _sc_mesh():
    return plsc.VectorSubcoreMesh(core_axis_name="c", subcore_axis_name="s")


_SC_PARAMS = pltpu.CompilerParams(use_tc_tiling_on_sc=False)


def _deg_body(edges, out, acc, ibuf, obuf, zbuf):
    c = lax.axis_index("c")
    s = lax.axis_index("s")
    zero16 = jnp.zeros((16,), jnp.float32)
    one16 = jnp.ones((16,), jnp.float32)

    def fill(i, _):
        zbuf[i] = zero16
        return 0

    lax.fori_loop(0, NZ, fill, 0)

    def fill1(i, _):
        obuf[i] = one16
        return 0

    lax.fori_loop(0, CW, fill1, 0)

    node_base = s * NODE_ROWS_PER_SUB

    def zcopy(k, _):
        pltpu.sync_copy(zbuf, acc.at[pl.ds(node_base + NZ * k, NZ)])
        return 0

    lax.fori_loop(0, NODE_ROWS_PER_SUB // NZ, zcopy, 0)
    plsc.subcore_barrier()

    def win(w, _):
        r0 = s * ROWS_PER_SUB + w * CH
        pltpu.sync_copy(edges.at[c, pl.ds(r0, CH)], ibuf)
        for j in range(CH):
            pltpu.sync_copy(obuf, acc.at[ibuf.at[j]], add=True)
        return 0

    lax.fori_loop(0, WINS, win, 0)
    plsc.subcore_barrier()

    cN = c * N

    def ocopy(k, _):
        r0 = node_base + NZ * k
        pltpu.sync_copy(acc.at[pl.ds(r0, NZ)], zbuf)
        pltpu.sync_copy(zbuf, out.at[pl.ds(cN + r0, NZ)])
        return 0

    lax.fori_loop(0, NODE_ROWS_PER_SUB // NZ, ocopy, 0)


_deg_call = pl.kernel(
    _deg_body,
    out_type=jax.ShapeDtypeStruct((2 * N, 16), jnp.float32),
    mesh=_sc_mesh(),
    scratch_types=[
        pltpu.VMEM_SHARED((N, 16), jnp.float32),
        pltpu.VMEM((CH, CW), jnp.int32),
        pltpu.VMEM((CW, 16), jnp.float32),
        pltpu.VMEM((NZ, 16), jnp.float32),
    ],
    compiler_params=_SC_PARAMS,
)


def _agg_body(tab, src2, dst2, out, acc, sbuf, dbuf, rbuf, zbuf, sem):
    c = lax.axis_index("c")
    s = lax.axis_index("s")
    zero16 = jnp.zeros((16,), jnp.float32)

    def fill(i, _):
        zbuf[i] = zero16
        return 0

    lax.fori_loop(0, NZ, fill, 0)

    node_base = s * NODE_ROWS_PER_SUB

    def zcopy(k, _):
        pltpu.sync_copy(zbuf, acc.at[pl.ds(node_base + NZ * k, NZ)])
        return 0

    lax.fori_loop(0, NODE_ROWS_PER_SUB // NZ, zcopy, 0)
    plsc.subcore_barrier()

    cN = c * N

    def win(w, _):
        r0 = s * ROWS_PER_SUB + w * CH
        pltpu.sync_copy(src2.at[pl.ds(r0, CH)], sbuf)
        pltpu.sync_copy(dst2.at[pl.ds(r0, CH)], dbuf)
        for j in range(CH):
            for k in range(CW // 16):
                sl = pl.ds(k * 16, 16)
                sbuf[j, sl] = sbuf[j, sl] + cN
        cps = [
            pltpu.async_copy(tab.at[sbuf.at[j]], rbuf.at[j], sem)
            for j in range(CH)
        ]
        for cp in cps:
            cp.wait()
        for j in range(CH):
            pltpu.sync_copy(rbuf.at[j], acc.at[dbuf.at[j]], add=True)
        return 0

    lax.fori_loop(0, WINS, win, 0)
    plsc.subcore_barrier()

    def ocopy(k, _):
        r0 = node_base + NZ * k
        pltpu.sync_copy(acc.at[pl.ds(r0, NZ)], zbuf)
        pltpu.sync_copy(zbuf, out.at[pl.ds(cN + r0, NZ)])
        return 0

    lax.fori_loop(0, NODE_ROWS_PER_SUB // NZ, ocopy, 0)


_agg_call = pl.kernel(
    _agg_body,
    out_type=jax.ShapeDtypeStruct((2 * N, 16), jnp.float32),
    mesh=_sc_mesh(),
    scratch_types=[
        pltpu.VMEM_SHARED((N, 16), jnp.float32),
        pltpu.VMEM((CH, CW), jnp.int32),
        pltpu.VMEM((CH, CW), jnp.int32),
        pltpu.VMEM((CH, CW, 16), jnp.float32),
        pltpu.VMEM((NZ, 16), jnp.float32),
        pltpu.SemaphoreType.DMA,
    ],
    compiler_params=_SC_PARAMS,
)


_BR = 1000
_GRID = N // _BR


def _prep_kernel(x_ref, cnt_ref, h_ref, ns_ref, ni_ref):
    cnt_s = cnt_ref[0][:, 0:1]
    cnt_i = cnt_ref[1][:, 0:1]
    ns = lax.rsqrt(cnt_s + 1.0)
    ni = lax.rsqrt(cnt_i + 1.0)
    ns_ref[...] = ns
    ni_ref[...] = ni
    h_ref[0] = x_ref[...] * ns
    h_ref[1] = jnp.zeros_like(x_ref)


def _prep(x, cnt):
    return pl.pallas_call(
        _prep_kernel,
        grid=(_GRID,),
        in_specs=[
            pl.BlockSpec((_BR, 16), lambda i: (i, 0)),
            pl.BlockSpec((2, _BR, 16), lambda i: (0, i, 0)),
        ],
        out_specs=[
            pl.BlockSpec((2, _BR, 16), lambda i: (0, i, 0)),
            pl.BlockSpec((_BR, 1), lambda i: (i, 0)),
            pl.BlockSpec((_BR, 1), lambda i: (i, 0)),
        ],
        out_shape=[
            jax.ShapeDtypeStruct((2, N, 16), jnp.float32),
            jax.ShapeDtypeStruct((N, 1), jnp.float32),
            jax.ShapeDtypeStruct((N, 1), jnp.float32),
        ],
    )(x, cnt)


def _dense_kernel(final, acc_ref, h_ref, ns_ref, ni_ref, w_ref, b_ref, o_ref):
    agg = jnp.concatenate(
        [acc_ref[0] + h_ref[0], acc_ref[1] + h_ref[1]], axis=1
    )
    z = jnp.dot(agg, w_ref[...], preferred_element_type=jnp.float32)
    z = z * ni_ref[...] + b_ref[...]
    if final:
        o_ref[...] = z[:, :20]
    else:
        h = jnp.maximum(z, 0.0) * ns_ref[...]
        o_ref[0] = h[:, :16]
        o_ref[1] = h[:, 16:32]


def _dense(final, acc, h, ns, ni, w32, b32):
    if final:
        out_spec = pl.BlockSpec((_BR, 20), lambda i: (i, 0))
        out_shape = jax.ShapeDtypeStruct((N, 20), jnp.float32)
    else:
        out_spec = pl.BlockSpec((2, _BR, 16), lambda i: (0, i, 0))
        out_shape = jax.ShapeDtypeStruct((2, N, 16), jnp.float32)
    return pl.pallas_call(
        functools.partial(_dense_kernel, final),
        grid=(_GRID,),
        in_specs=[
            pl.BlockSpec((2, _BR, 16), lambda i: (0, i, 0)),
            pl.BlockSpec((2, _BR, 16), lambda i: (0, i, 0)),
            pl.BlockSpec((_BR, 1), lambda i: (i, 0)),
            pl.BlockSpec((_BR, 1), lambda i: (i, 0)),
            pl.BlockSpec((32, 32), lambda i: (0, 0)),
            pl.BlockSpec((1, 32), lambda i: (0, 0)),
        ],
        out_specs=out_spec,
        out_shape=out_shape,
    )(acc, h, ns, ni, w32, b32)


def _pad_w(w):
    return jnp.zeros((32, 32), jnp.float32).at[: w.shape[0], : w.shape[1]].set(w)


def _pad_b(b):
    return jnp.zeros((1, 32), jnp.float32).at[0, : b.shape[0]].set(b)


def kernel(x, edge_index, W0, b0, Wh, bh, Wf, bf):
    assert x.shape == (N, 16) and edge_index.shape == (2, E)
    edges_st = edge_index.reshape(2, EROWS, CW)
    src2 = edges_st[0]
    dst2 = edges_st[1]

    cnt = _deg_call(edges_st).reshape(2, N, 16)
    hpair, ns, ni = _prep(x, cnt)

    ws = [_pad_w(W0)] + [_pad_w(Wh[i]) for i in range(Wh.shape[0])] + [_pad_w(Wf)]
    bs = [_pad_b(b0)] + [_pad_b(bh[i]) for i in range(bh.shape[0])] + [_pad_b(bf)]

    n_layers = len(ws)
    for li in range(n_layers):
        tab = hpair.reshape(2 * N, 16)
        acc = _agg_call(tab, src2, dst2).reshape(2, N, 16)
        final = li == n_layers - 1
        if final:
            out = _dense(True, acc, hpair, ns, ni, ws[li], bs[li])
        else:
            hpair = _dense(False, acc, hpair, ns, ni, ws[li], bs[li])
    return out[None]


# speedup vs baseline: 12.4895x; 1.1705x over previous
import functools

import jax
import jax.numpy as jnp
from jax import lax
from jax.experimental import pallas as pl
from jax.experimental.pallas import tpu as pltpu
from jax.experimental.pallas import tpu_sc as plsc

N = 100000
E = 3200000
NC = 2
NS = 16
CW = 80
CH = 20
EROWS = E // CW
ROWS_PER_SUB = EROWS // NS
WINS = ROWS_PER_SUB // CH
NZ = 125
NODE_ROWS_PER_SUB = N // NS


def _sc_mesh():
    return plsc.VectorSubcoreMesh(core_axis_name="c", subcore_axis_name="s")


_SC_PARAMS = pltpu.CompilerParams(use_tc_tiling_on_sc=False)


def _deg_body(edges, out, acc, ibuf, obuf, zbuf):
    c = lax.axis_index("c")
    s = lax.axis_index("s")
    zero16 = jnp.zeros((16,), jnp.float32)
    one16 = jnp.ones((16,), jnp.float32)

    def fill(i, _):
        zbuf[i] = zero16
        return 0

    lax.fori_loop(0, NZ, fill, 0)

    def fill1(i, _):
        obuf[i] = one16
        return 0

    lax.fori_loop(0, CW, fill1, 0)

    node_base = s * NODE_ROWS_PER_SUB

    def zcopy(k, _):
        pltpu.sync_copy(zbuf, acc.at[pl.ds(node_base + NZ * k, NZ)])
        return 0

    lax.fori_loop(0, NODE_ROWS_PER_SUB // NZ, zcopy, 0)
    plsc.subcore_barrier()

    def win(w, _):
        r0 = s * ROWS_PER_SUB + w * CH
        pltpu.sync_copy(edges.at[c, pl.ds(r0, CH)], ibuf)
        for j in range(CH):
            pltpu.sync_copy(obuf, acc.at[ibuf.at[j]], add=True)
        return 0

    lax.fori_loop(0, WINS, win, 0)
    plsc.subcore_barrier()

    cN = c * N

    def ocopy(k, _):
        r0 = node_base + NZ * k
        pltpu.sync_copy(acc.at[pl.ds(r0, NZ)], zbuf)
        pltpu.sync_copy(zbuf, out.at[pl.ds(cN + r0, NZ)])
        return 0

    lax.fori_loop(0, NODE_ROWS_PER_SUB // NZ, ocopy, 0)


_deg_call = pl.kernel(
    _deg_body,
    out_type=jax.ShapeDtypeStruct((2 * N, 16), jnp.float32),
    mesh=_sc_mesh(),
    scratch_types=[
        pltpu.VMEM_SHARED((N, 16), jnp.float32),
        pltpu.VMEM((CH, CW), jnp.int32),
        pltpu.VMEM((CW, 16), jnp.float32),
        pltpu.VMEM((NZ, 16), jnp.float32),
    ],
    compiler_params=_SC_PARAMS,
)


def _agg_body(tab, src2, dst2, out, acc, sbuf, dbuf, rbuf, zbuf, sem):
    c = lax.axis_index("c")
    s = lax.axis_index("s")
    zero16 = jnp.zeros((16,), jnp.float32)

    def fill(i, _):
        zbuf[i] = zero16
        return 0

    lax.fori_loop(0, NZ, fill, 0)

    node_base = s * NODE_ROWS_PER_SUB

    def zcopy(k, _):
        pltpu.sync_copy(zbuf, acc.at[pl.ds(node_base + NZ * k, NZ)])
        return 0

    lax.fori_loop(0, NODE_ROWS_PER_SUB // NZ, zcopy, 0)
    plsc.subcore_barrier()

    cN = c * N

    def win(w, _):
        r0 = s * ROWS_PER_SUB + w * CH
        pltpu.sync_copy(src2.at[pl.ds(r0, CH)], sbuf)
        pltpu.sync_copy(dst2.at[pl.ds(r0, CH)], dbuf)
        for j in range(CH):
            for k in range(CW // 16):
                sl = pl.ds(k * 16, 16)
                sbuf[j, sl] = sbuf[j, sl] + cN
        cps = [
            pltpu.async_copy(tab.at[sbuf.at[j]], rbuf.at[j], sem)
            for j in range(CH)
        ]
        for cp in cps:
            cp.wait()
        for j in range(CH):
            pltpu.sync_copy(rbuf.at[j], acc.at[dbuf.at[j]], add=True)
        return 0

    lax.fori_loop(0, WINS, win, 0)
    plsc.subcore_barrier()

    def ocopy(k, _):
        r0 = node_base + NZ * k
        pltpu.sync_copy(acc.at[pl.ds(r0, NZ)], zbuf)
        pltpu.sync_copy(zbuf, out.at[pl.ds(cN + r0, NZ)])
        return 0

    lax.fori_loop(0, NODE_ROWS_PER_SUB // NZ, ocopy, 0)


_agg_call = pl.kernel(
    _agg_body,
    out_type=jax.ShapeDtypeStruct((2 * N, 16), jnp.float32),
    mesh=_sc_mesh(),
    scratch_types=[
        pltpu.VMEM_SHARED((N, 16), jnp.float32),
        pltpu.VMEM((CH, CW), jnp.int32),
        pltpu.VMEM((CH, CW), jnp.int32),
        pltpu.VMEM((CH, CW, 16), jnp.float32),
        pltpu.VMEM((NZ, 16), jnp.float32),
        pltpu.SemaphoreType.DMA,
    ],
    compiler_params=_SC_PARAMS,
)


_BR = 1000
_GRID = N // _BR


def _prep_kernel(x_ref, cnt_ref, h_ref, ns_ref, ni_ref):
    cnt_s = cnt_ref[0][:, 0:1]
    cnt_i = cnt_ref[1][:, 0:1]
    ns = lax.rsqrt(cnt_s + 1.0)
    ni = lax.rsqrt(cnt_i + 1.0)
    ns_ref[...] = ns
    ni_ref[...] = ni
    h_ref[0] = x_ref[...] * ns
    h_ref[1] = jnp.zeros_like(x_ref)


def _prep(x, cnt):
    return pl.pallas_call(
        _prep_kernel,
        grid=(_GRID,),
        in_specs=[
            pl.BlockSpec((_BR, 16), lambda i: (i, 0)),
            pl.BlockSpec((2, _BR, 16), lambda i: (0, i, 0)),
        ],
        out_specs=[
            pl.BlockSpec((2, _BR, 16), lambda i: (0, i, 0)),
            pl.BlockSpec((_BR, 1), lambda i: (i, 0)),
            pl.BlockSpec((_BR, 1), lambda i: (i, 0)),
        ],
        out_shape=[
            jax.ShapeDtypeStruct((2, N, 16), jnp.float32),
            jax.ShapeDtypeStruct((N, 1), jnp.float32),
            jax.ShapeDtypeStruct((N, 1), jnp.float32),
        ],
    )(x, cnt)


def _dense_kernel(final, acc_ref, h_ref, ns_ref, ni_ref, w_ref, b_ref, o_ref):
    agg = jnp.concatenate(
        [acc_ref[0] + h_ref[0], acc_ref[1] + h_ref[1]], axis=1
    )
    z = jnp.dot(agg, w_ref[...], preferred_element_type=jnp.float32)
    z = z * ni_ref[...] + b_ref[...]
    if final:
        o_ref[...] = z[:, :20]
    else:
        h = jnp.maximum(z, 0.0) * ns_ref[...]
        o_ref[0] = h[:, :16]
        o_ref[1] = h[:, 16:32]


def _dense(final, acc, h, ns, ni, w32, b32):
    if final:
        out_spec = pl.BlockSpec((_BR, 20), lambda i: (i, 0))
        out_shape = jax.ShapeDtypeStruct((N, 20), jnp.float32)
    else:
        out_spec = pl.BlockSpec((2, _BR, 16), lambda i: (0, i, 0))
        out_shape = jax.ShapeDtypeStruct((2, N, 16), jnp.float32)
    return pl.pallas_call(
        functools.partial(_dense_kernel, final),
        grid=(_GRID,),
        in_specs=[
            pl.BlockSpec((2, _BR, 16), lambda i: (0, i, 0)),
            pl.BlockSpec((2, _BR, 16), lambda i: (0, i, 0)),
            pl.BlockSpec((_BR, 1), lambda i: (i, 0)),
            pl.BlockSpec((_BR, 1), lambda i: (i, 0)),
            pl.BlockSpec((32, 32), lambda i: (0, 0)),
            pl.BlockSpec((1, 32), lambda i: (0, 0)),
        ],
        out_specs=out_spec,
        out_shape=out_shape,
    )(acc, h, ns, ni, w32, b32)


def _pad_w(w):
    return jnp.zeros((32, 32), jnp.float32).at[: w.shape[0], : w.shape[1]].set(w)


def _pad_b(b):
    return jnp.zeros((1, 32), jnp.float32).at[0, : b.shape[0]].set(b)


def kernel(x, edge_index, W0, b0, Wh, bh, Wf, bf):
    assert x.shape == (N, 16) and edge_index.shape == (2, E)
    edges_st = edge_index.reshape(2, EROWS, CW)
    src2 = edges_st[0]
    dst2 = edges_st[1]

    cnt = _deg_call(edges_st).reshape(2, N, 16)
    hpair, ns, ni = _prep(x, cnt)

    ws = [_pad_w(W0)] + [_pad_w(Wh[i]) for i in range(Wh.shape[0])] + [_pad_w(Wf)]
    bs = [_pad_b(b0)] + [_pad_b(bh[i]) for i in range(bh.shape[0])] + [_pad_b(bf)]

    n_layers = len(ws)
    for li in range(n_layers):
        tab = hpair.reshape(2 * N, 16)
        acc = _agg_call(tab, src2, dst2).reshape(2, N, 16)
        final = li == n_layers - 1
        if final:
            out = _dense(True, acc, hpair, ns, ni, ws[li], bs[li])
        else:
            hpair = _dense(False, acc, hpair, ns, ni, ws[li], bs[li])
    return out[None]


# speedup vs baseline: 16.3627x; 1.3101x over previous
import functools

import jax
import jax.numpy as jnp
from jax import lax
from jax.experimental import pallas as pl
from jax.experimental.pallas import tpu as pltpu
from jax.experimental.pallas import tpu_sc as plsc

N = 100000
E = 3200000
NC = 2
NS = 16
CW = 80
CH = 10
EROWS = E // CW
ROWS_PER_SUB = EROWS // NS
WINS = ROWS_PER_SUB // CH
NZ = 125
NODE_ROWS_PER_SUB = N // NS


def _sc_mesh():
    return plsc.VectorSubcoreMesh(core_axis_name="c", subcore_axis_name="s")


_SC_PARAMS = pltpu.CompilerParams(use_tc_tiling_on_sc=False)


def _deg_body(edges, out, acc, ibuf, obuf, zbuf):
    c = lax.axis_index("c")
    s = lax.axis_index("s")
    zero16 = jnp.zeros((16,), jnp.float32)
    one16 = jnp.ones((16,), jnp.float32)

    def fill(i, _):
        zbuf[i] = zero16
        return 0

    lax.fori_loop(0, NZ, fill, 0)

    def fill1(i, _):
        obuf[i] = one16
        return 0

    lax.fori_loop(0, CW, fill1, 0)

    node_base = s * NODE_ROWS_PER_SUB

    def zcopy(k, _):
        pltpu.sync_copy(zbuf, acc.at[pl.ds(node_base + NZ * k, NZ)])
        return 0

    lax.fori_loop(0, NODE_ROWS_PER_SUB // NZ, zcopy, 0)
    plsc.subcore_barrier()

    def win(w, _):
        r0 = s * ROWS_PER_SUB + w * CH
        pltpu.sync_copy(edges.at[c, pl.ds(r0, CH)], ibuf)
        for j in range(CH):
            pltpu.sync_copy(obuf, acc.at[ibuf.at[j]], add=True)
        return 0

    lax.fori_loop(0, WINS, win, 0)
    plsc.subcore_barrier()

    cN = c * N

    def ocopy(k, _):
        r0 = node_base + NZ * k
        pltpu.sync_copy(acc.at[pl.ds(r0, NZ)], zbuf)
        pltpu.sync_copy(zbuf, out.at[pl.ds(cN + r0, NZ)])
        return 0

    lax.fori_loop(0, NODE_ROWS_PER_SUB // NZ, ocopy, 0)


_deg_call = pl.kernel(
    _deg_body,
    out_type=jax.ShapeDtypeStruct((2 * N, 16), jnp.float32),
    mesh=_sc_mesh(),
    scratch_types=[
        pltpu.VMEM_SHARED((N, 16), jnp.float32),
        pltpu.VMEM((CH, CW), jnp.int32),
        pltpu.VMEM((CW, 16), jnp.float32),
        pltpu.VMEM((NZ, 16), jnp.float32),
    ],
    compiler_params=_SC_PARAMS,
)


def _agg_body(tab, srcs, dst2, out, acc,
              sbufA, sbufB, dbufA, dbufB, rbufA, rbufB, zbuf, semg, semsc):
    c = lax.axis_index("c")
    s = lax.axis_index("s")
    zero16 = jnp.zeros((16,), jnp.float32)

    def fill(i, _):
        zbuf[i] = zero16
        return 0

    lax.fori_loop(0, NZ, fill, 0)

    node_base = s * NODE_ROWS_PER_SUB

    def zcopy(k, _):
        pltpu.sync_copy(zbuf, acc.at[pl.ds(node_base + NZ * k, NZ)])
        return 0

    lax.fori_loop(0, NODE_ROWS_PER_SUB // NZ, zcopy, 0)
    plsc.subcore_barrier()

    cN = c * N
    row_base = s * ROWS_PER_SUB

    def load_idx(w, sb, db):
        r0 = row_base + w * CH
        pltpu.sync_copy(srcs.at[c, pl.ds(r0, CH)], sb)
        pltpu.sync_copy(dst2.at[pl.ds(r0, CH)], db)

    def fire_gathers(sb, rb):
        for j in range(CH):
            pltpu.async_copy(tab.at[sb.at[j]], rb.at[j], semg)

    def wait_gathers(sb, rb):
        for j in range(CH):
            pltpu.make_async_copy(tab.at[sb.at[j]], rb.at[j], semg).wait()

    def fire_scatters(rb, db):
        return [
            pltpu.async_copy(rb.at[j], acc.at[db.at[j]], semsc, add=True)
            for j in range(CH)
        ]

    load_idx(0, sbufA, dbufA)
    fire_gathers(sbufA, rbufA)

    def pair(i, _):
        load_idx(2 * i + 1, sbufB, dbufB)
        wait_gathers(sbufA, rbufA)
        fire_gathers(sbufB, rbufB)
        csA = fire_scatters(rbufA, dbufA)
        for cp in csA:
            cp.wait()
        wA = lax.rem(2 * i + 2, WINS)
        load_idx(wA, sbufA, dbufA)
        wait_gathers(sbufB, rbufB)
        fire_gathers(sbufA, rbufA)
        csB = fire_scatters(rbufB, dbufB)
        for cp in csB:
            cp.wait()
        return 0

    lax.fori_loop(0, WINS // 2, pair, 0)
    wait_gathers(sbufA, rbufA)
    plsc.subcore_barrier()

    def ocopy(k, _):
        r0 = node_base + NZ * k
        pltpu.sync_copy(acc.at[pl.ds(r0, NZ)], zbuf)
        pltpu.sync_copy(zbuf, out.at[pl.ds(cN + r0, NZ)])
        return 0

    lax.fori_loop(0, NODE_ROWS_PER_SUB // NZ, ocopy, 0)


_agg_call = pl.kernel(
    _agg_body,
    out_type=jax.ShapeDtypeStruct((2 * N, 16), jnp.float32),
    mesh=_sc_mesh(),
    scratch_types=[
        pltpu.VMEM_SHARED((N, 16), jnp.float32),
        pltpu.VMEM((CH, CW), jnp.int32),
        pltpu.VMEM((CH, CW), jnp.int32),
        pltpu.VMEM((CH, CW), jnp.int32),
        pltpu.VMEM((CH, CW), jnp.int32),
        pltpu.VMEM((CH, CW, 16), jnp.float32),
        pltpu.VMEM((CH, CW, 16), jnp.float32),
        pltpu.VMEM((NZ, 16), jnp.float32),
        pltpu.SemaphoreType.DMA,
        pltpu.SemaphoreType.DMA,
    ],
    compiler_params=_SC_PARAMS,
)


_BR = 1000
_GRID = N // _BR


def _prep_kernel(x_ref, cnt_ref, h_ref, ns_ref, ni_ref):
    cnt_s = cnt_ref[0][:, 0:1]
    cnt_i = cnt_ref[1][:, 0:1]
    ns = lax.rsqrt(cnt_s + 1.0)
    ni = lax.rsqrt(cnt_i + 1.0)
    ns_ref[...] = ns
    ni_ref[...] = ni
    h_ref[0] = x_ref[...] * ns
    h_ref[1] = jnp.zeros_like(x_ref)


def _prep(x, cnt):
    return pl.pallas_call(
        _prep_kernel,
        grid=(_GRID,),
        in_specs=[
            pl.BlockSpec((_BR, 16), lambda i: (i, 0)),
            pl.BlockSpec((2, _BR, 16), lambda i: (0, i, 0)),
        ],
        out_specs=[
            pl.BlockSpec((2, _BR, 16), lambda i: (0, i, 0)),
            pl.BlockSpec((_BR, 1), lambda i: (i, 0)),
            pl.BlockSpec((_BR, 1), lambda i: (i, 0)),
        ],
        out_shape=[
            jax.ShapeDtypeStruct((2, N, 16), jnp.float32),
            jax.ShapeDtypeStruct((N, 1), jnp.float32),
            jax.ShapeDtypeStruct((N, 1), jnp.float32),
        ],
    )(x, cnt)


def _dense_kernel(final, acc_ref, h_ref, ns_ref, ni_ref, w_ref, b_ref, o_ref):
    agg = jnp.concatenate(
        [acc_ref[0] + h_ref[0], acc_ref[1] + h_ref[1]], axis=1
    )
    z = jnp.dot(agg, w_ref[...], preferred_element_type=jnp.float32)
    z = z * ni_ref[...] + b_ref[...]
    if final:
        o_ref[...] = z[:, :20]
    else:
        h = jnp.maximum(z, 0.0) * ns_ref[...]
        o_ref[0] = h[:, :16]
        o_ref[1] = h[:, 16:32]


def _dense(final, acc, h, ns, ni, w32, b32):
    if final:
        out_spec = pl.BlockSpec((_BR, 20), lambda i: (i, 0))
        out_shape = jax.ShapeDtypeStruct((N, 20), jnp.float32)
    else:
        out_spec = pl.BlockSpec((2, _BR, 16), lambda i: (0, i, 0))
        out_shape = jax.ShapeDtypeStruct((2, N, 16), jnp.float32)
    return pl.pallas_call(
        functools.partial(_dense_kernel, final),
        grid=(_GRID,),
        in_specs=[
            pl.BlockSpec((2, _BR, 16), lambda i: (0, i, 0)),
            pl.BlockSpec((2, _BR, 16), lambda i: (0, i, 0)),
            pl.BlockSpec((_BR, 1), lambda i: (i, 0)),
            pl.BlockSpec((_BR, 1), lambda i: (i, 0)),
            pl.BlockSpec((32, 32), lambda i: (0, 0)),
            pl.BlockSpec((1, 32), lambda i: (0, 0)),
        ],
        out_specs=out_spec,
        out_shape=out_shape,
    )(acc, h, ns, ni, w32, b32)


def _pad_w(w):
    return jnp.zeros((32, 32), jnp.float32).at[: w.shape[0], : w.shape[1]].set(w)


def _pad_b(b):
    return jnp.zeros((1, 32), jnp.float32).at[0, : b.shape[0]].set(b)


def kernel(x, edge_index, W0, b0, Wh, bh, Wf, bf):
    assert x.shape == (N, 16) and edge_index.shape == (2, E)
    edges_st = edge_index.reshape(2, EROWS, CW)
    src2 = edges_st[0]
    dst2 = edges_st[1]
    srcs = jnp.stack([src2, src2 + N])

    cnt = _deg_call(edges_st).reshape(2, N, 16)
    hpair, ns, ni = _prep(x, cnt)

    ws = [_pad_w(W0)] + [_pad_w(Wh[i]) for i in range(Wh.shape[0])] + [_pad_w(Wf)]
    bs = [_pad_b(b0)] + [_pad_b(bh[i]) for i in range(bh.shape[0])] + [_pad_b(bf)]

    n_layers = len(ws)
    for li in range(n_layers):
        tab = hpair.reshape(2 * N, 16)
        acc = _agg_call(tab, srcs, dst2).reshape(2, N, 16)
        final = li == n_layers - 1
        if final:
            out = _dense(True, acc, hpair, ns, ni, ws[li], bs[li])
        else:
            hpair = _dense(False, acc, hpair, ns, ni, ws[li], bs[li])
    return out[None]
